# baseline (speedup 1.0000x reference)
import jax
import jax.numpy as jnp
from jax import lax
from jax.experimental import pallas as pl
from jax.experimental.pallas import tpu as pltpu

N = 10000
D = 256
BM = 200
NSTEP = N // BM
NBUF = 3
LAST = NSTEP - 1
LSLOT = LAST % NBUF

KCH = ((0, 2560), (2560, 2560), (5120, 2560), (7680, 2320))


def _dot(a, xb):
    return jax.lax.dot_general(a, xb, (((1,), (0,)), ((), ())),
                               preferred_element_type=jnp.float32)


def _qcopy(adj_ref, abufs, sems, j, slot, c):
    off, sz = KCH[c]
    return pltpu.make_async_copy(
        adj_ref.at[pl.ds(j * BM, BM), pl.ds(off, sz)],
        abufs.at[slot, :, pl.ds(off, sz)], sems.at[slot, c])


def _issue(adj_ref, abufs, sems, j):
    slot = lax.rem(j, NBUF)
    for c in range(len(KCH)):
        _qcopy(adj_ref, abufs, sems, j, slot, c).start()


def _gcn_block(adj_ref, x_ref, o_ref, abufs, xf, xb, sems, xsems):
    i = pl.program_id(0)
    slot = lax.rem(i, NBUF)

    @pl.when(i == 0)
    def _():
        _issue(adj_ref, abufs, sems, 0)
        for c, (off, sz) in enumerate(KCH):
            pltpu.make_async_copy(x_ref.at[pl.ds(off, sz), :],
                                  xf.at[pl.ds(off, sz), :], xsems.at[c]).start()
        _issue(adj_ref, abufs, sems, 1)
        acc = jnp.zeros((BM, D), jnp.float32)
        for c, (off, sz) in enumerate(KCH):
            pltpu.make_async_copy(x_ref.at[pl.ds(off, sz), :],
                                  xf.at[pl.ds(off, sz), :], xsems.at[c]).wait()
            xb[pl.ds(off, sz), :] = xf[pl.ds(off, sz), :].astype(jnp.bfloat16)
            _qcopy(adj_ref, abufs, sems, 0, 0, c).wait()
            acc += _dot(abufs[0, :, pl.ds(off, sz)], xb[pl.ds(off, sz), :])
        o_ref[...] = acc

    @pl.when(i + NBUF - 1 < NSTEP)
    def _():
        _issue(adj_ref, abufs, sems, i + NBUF - 1)

    @pl.when((i > 0) & (i < LAST))
    def _():
        for c in range(len(KCH)):
            _qcopy(adj_ref, abufs, sems, i, slot, c).wait()
        o_ref[...] = _dot(abufs[slot], xb[...])

    @pl.when(i == LAST)
    def _():
        acc = jnp.zeros((BM, D), jnp.float32)
        for c, (off, sz) in enumerate(KCH):
            _qcopy(adj_ref, abufs, sems, LAST, LSLOT, c).wait()
            acc += _dot(abufs[LSLOT, :, pl.ds(off, sz)], xb[pl.ds(off, sz), :])
        o_ref[...] = acc


@jax.jit
def kernel(adj, embeds):
    return pl.pallas_call(
        _gcn_block,
        grid=(NSTEP,),
        in_specs=[
            pl.BlockSpec(memory_space=pltpu.MemorySpace.HBM),
            pl.BlockSpec(memory_space=pltpu.MemorySpace.HBM),
        ],
        out_specs=pl.BlockSpec((BM, D), lambda i: (i, 0)),
        out_shape=jax.ShapeDtypeStruct((N, D), jnp.float32),
        scratch_shapes=[
            pltpu.VMEM((NBUF, BM, N), jnp.float32),
            pltpu.VMEM((N, D), jnp.float32),
            pltpu.VMEM((N, D), jnp.bfloat16),
            pltpu.SemaphoreType.DMA((NBUF, len(KCH))),
            pltpu.SemaphoreType.DMA((len(KCH),)),
        ],
        compiler_params=pltpu.CompilerParams(
            dimension_semantics=("arbitrary",),
        ),
    )(adj, embeds)


# speedup vs baseline: 1.0115x; 1.0115x over previous
import jax
import jax.numpy as jnp
from jax import lax
from jax.experimental import pallas as pl
from jax.experimental.pallas import tpu as pltpu

N = 10000
D = 256
BM = 200
NSTEP = N // BM
NBUF = 3
LAST = NSTEP - 1
LSLOT = LAST % NBUF

XCH = ((0, 2560), (2560, 2560), (5120, 2560), (7680, 2320))
KSPLIT = 5120
KH = ((0, KSPLIT), (KSPLIT, N - KSPLIT))


def _dot(a, xb):
    return jax.lax.dot_general(a, xb, (((1,), (0,)), ((), ())),
                               preferred_element_type=jnp.float32)


def _half_copy(adj_ref, abufs, sems, j, slot, h):
    off, sz = KH[h]
    return pltpu.make_async_copy(
        adj_ref.at[pl.ds(j * BM, BM), pl.ds(off, sz)],
        abufs.at[slot, :, pl.ds(off, sz)], sems.at[slot, h])


def _issue(adj_ref, abufs, sems, j):
    slot = lax.rem(j, NBUF)
    for h in range(2):
        _half_copy(adj_ref, abufs, sems, j, slot, h).start()


def _gcn_block(adj_ref, x_ref, o_ref, abufs, xf, xb, sems, xsems):
    i = pl.program_id(0)
    slot = lax.rem(i, NBUF)

    @pl.when(i == 0)
    def _():
        _issue(adj_ref, abufs, sems, 0)
        for c, (off, sz) in enumerate(XCH):
            pltpu.make_async_copy(x_ref.at[pl.ds(off, sz), :],
                                  xf.at[pl.ds(off, sz), :], xsems.at[c]).start()
        _issue(adj_ref, abufs, sems, 1)
        acc = jnp.zeros((BM, D), jnp.float32)
        for c, (off, sz) in enumerate(XCH):
            pltpu.make_async_copy(x_ref.at[pl.ds(off, sz), :],
                                  xf.at[pl.ds(off, sz), :], xsems.at[c]).wait()
            xb[pl.ds(off, sz), :] = xf[pl.ds(off, sz), :].astype(jnp.bfloat16)
            if c == 0:
                _half_copy(adj_ref, abufs, sems, 0, 0, 0).wait()
            elif c == 2:
                _half_copy(adj_ref, abufs, sems, 0, 0, 1).wait()
            acc += _dot(abufs[0, :, pl.ds(off, sz)], xb[pl.ds(off, sz), :])
        o_ref[...] = acc

    @pl.when(i + NBUF - 1 < NSTEP)
    def _():
        _issue(adj_ref, abufs, sems, i + NBUF - 1)

    @pl.when((i > 0) & (i < LAST))
    def _():
        for h in range(2):
            _half_copy(adj_ref, abufs, sems, i, slot, h).wait()
        o_ref[...] = _dot(abufs[slot], xb[...])

    @pl.when(i == LAST)
    def _():
        _half_copy(adj_ref, abufs, sems, LAST, LSLOT, 0).wait()
        acc = _dot(abufs[LSLOT, :, pl.ds(0, KSPLIT)], xb[pl.ds(0, KSPLIT), :])
        _half_copy(adj_ref, abufs, sems, LAST, LSLOT, 1).wait()
        acc += _dot(abufs[LSLOT, :, pl.ds(KSPLIT, N - KSPLIT)],
                    xb[pl.ds(KSPLIT, N - KSPLIT), :])
        o_ref[...] = acc


@jax.jit
def kernel(adj, embeds):
    return pl.pallas_call(
        _gcn_block,
        grid=(NSTEP,),
        in_specs=[
            pl.BlockSpec(memory_space=pltpu.MemorySpace.HBM),
            pl.BlockSpec(memory_space=pltpu.MemorySpace.HBM),
        ],
        out_specs=pl.BlockSpec((BM, D), lambda i: (i, 0)),
        out_shape=jax.ShapeDtypeStruct((N, D), jnp.float32),
        scratch_shapes=[
            pltpu.VMEM((NBUF, BM, N), jnp.float32),
            pltpu.VMEM((N, D), jnp.float32),
            pltpu.VMEM((N, D), jnp.bfloat16),
            pltpu.SemaphoreType.DMA((NBUF, 2)),
            pltpu.SemaphoreType.DMA((len(XCH),)),
        ],
        compiler_params=pltpu.CompilerParams(
            dimension_semantics=("arbitrary",),
        ),
    )(adj, embeds)


# speedup vs baseline: 1.0132x; 1.0017x over previous
import jax
import jax.numpy as jnp
from jax import lax
from jax.experimental import pallas as pl
from jax.experimental.pallas import tpu as pltpu

N = 10000
D = 256
BM = 200
NSTEP = N // BM
NBUF = 3
LAST = NSTEP - 1
LSLOT = LAST % NBUF

XCH = ((0, 2560), (2560, 2560), (5120, 2560), (7680, 2320))
KSPLIT = 4992


def _dot(a, xb):
    return jax.lax.dot_general(a, xb, (((1,), (0,)), ((), ())),
                               preferred_element_type=jnp.float32)


def _full_copy(adj_ref, abufs, sems, j, slot):
    return pltpu.make_async_copy(
        adj_ref.at[pl.ds(j * BM, BM), :], abufs.at[slot], sems.at[slot])


def _half_copy(adj_ref, abufs, lsems, h):
    off, sz = ((0, KSPLIT), (KSPLIT, N - KSPLIT))[h]
    return pltpu.make_async_copy(
        adj_ref.at[pl.ds(LAST * BM, BM), pl.ds(off, sz)],
        abufs.at[LSLOT, :, pl.ds(off, sz)], lsems.at[h])


def _issue(adj_ref, abufs, sems, lsems, j):
    slot = lax.rem(j, NBUF)

    @pl.when(j < LAST)
    def _():
        _full_copy(adj_ref, abufs, sems, j, slot).start()

    @pl.when(j == LAST)
    def _():
        for h in range(2):
            _half_copy(adj_ref, abufs, lsems, h).start()


def _gcn_block(adj_ref, x_ref, o_ref, abufs, xf, xb, sems, xsems, lsems):
    i = pl.program_id(0)
    slot = lax.rem(i, NBUF)

    @pl.when(i == 0)
    def _():
        _full_copy(adj_ref, abufs, sems, 0, 0).start()
        for c, (off, sz) in enumerate(XCH):
            pltpu.make_async_copy(x_ref.at[pl.ds(off, sz), :],
                                  xf.at[pl.ds(off, sz), :], xsems.at[c]).start()
        _full_copy(adj_ref, abufs, sems, 1, 1).start()
        _full_copy(adj_ref, abufs, sems, 0, 0).wait()
        acc = jnp.zeros((BM, D), jnp.float32)
        for c, (off, sz) in enumerate(XCH):
            pltpu.make_async_copy(x_ref.at[pl.ds(off, sz), :],
                                  xf.at[pl.ds(off, sz), :], xsems.at[c]).wait()
            xb[pl.ds(off, sz), :] = xf[pl.ds(off, sz), :].astype(jnp.bfloat16)
            acc += _dot(abufs[0, :, pl.ds(off, sz)], xb[pl.ds(off, sz), :])
        o_ref[...] = acc

    _issue(adj_ref, abufs, sems, lsems, i + NBUF - 1)

    @pl.when((i > 0) & (i < LAST))
    def _():
        _full_copy(adj_ref, abufs, sems, i, slot).wait()
        o_ref[...] = _dot(abufs[slot], xb[...])

    @pl.when(i == LAST)
    def _():
        _half_copy(adj_ref, abufs, lsems, 0).wait()
        acc = _dot(abufs[LSLOT, :, pl.ds(0, KSPLIT)],
                   xb[pl.ds(0, KSPLIT), :])
        _half_copy(adj_ref, abufs, lsems, 1).wait()
        acc += _dot(abufs[LSLOT, :, pl.ds(KSPLIT, N - KSPLIT)],
                    xb[pl.ds(KSPLIT, N - KSPLIT), :])
        o_ref[...] = acc


@jax.jit
def kernel(adj, embeds):
    return pl.pallas_call(
        _gcn_block,
        grid=(NSTEP,),
        in_specs=[
            pl.BlockSpec(memory_space=pltpu.MemorySpace.HBM),
            pl.BlockSpec(memory_space=pltpu.MemorySpace.HBM),
        ],
        out_specs=pl.BlockSpec((BM, D), lambda i: (i, 0)),
        out_shape=jax.ShapeDtypeStruct((N, D), jnp.float32),
        scratch_shapes=[
            pltpu.VMEM((NBUF, BM, N), jnp.float32),
            pltpu.VMEM((N, D), jnp.float32),
            pltpu.VMEM((N, D), jnp.bfloat16),
            pltpu.SemaphoreType.DMA((NBUF,)),
            pltpu.SemaphoreType.DMA((len(XCH),)),
            pltpu.SemaphoreType.DMA((2,)),
        ],
        compiler_params=pltpu.CompilerParams(
            dimension_semantics=("arbitrary",),
        ),
    )(adj, embeds)
